# Initial kernel scaffold; baseline (speedup 1.0000x reference)
import jax
import jax.numpy as jnp
from jax.experimental import pallas as pl


def kernel(x, W_qkv, W_dw, W_proj, temperature, attn1p):
    raise NotImplementedError("write your pallas kernel here")


# speedup vs baseline: 4.2231x; 4.2231x over previous
import jax
import jax.numpy as jnp
from jax.experimental import pallas as pl
from jax.experimental.pallas import tpu as pltpu

DIM = 192
HEADS = 6
C = DIM // HEADS
H = 224
W = 224
HWTOT = H * W

R1 = 8
T1 = H // R1
N1 = R1 * W
R2 = 16
T2 = H // R2
N2 = R2 * W
R4 = 16
T4 = H // R4
N4 = R4 * W

_TAPS = [(dy, dx) for dy in (-1, 0, 1) for dx in (-1, 0, 1)]


def _p1_body(x_top_ref, x_cur_ref, x_bot_ref, w1_ref, wdw_ref,
             z_ref, qsq_ref, ksq_ref):
    i = pl.program_id(0)
    zero_row = jnp.zeros((DIM, W), jnp.float32)
    top = jnp.where(i == 0, zero_row, x_top_ref[:, 7, :])
    bot = jnp.where(i == T1 - 1, zero_row, x_bot_ref[:, 0, :])
    x_ext = jnp.concatenate([top, x_cur_ref[...], bot], axis=1)

    y_ext = jax.lax.dot_general(
        w1_ref[...].astype(jnp.bfloat16), x_ext.astype(jnp.bfloat16),
        (((1,), (0,)), ((), ())), preferred_element_type=jnp.float32)
    y_ext = y_ext.astype(jnp.bfloat16).astype(jnp.float32)

    widx = jax.lax.broadcasted_iota(jnp.int32, (1, N1), 1) % W
    mask_l = (widx != 0).astype(jnp.float32)
    mask_r = (widx != W - 1).astype(jnp.float32)
    zcol = jnp.zeros((3 * DIM, 1), jnp.float32)
    y_pad = jnp.concatenate([zcol, y_ext, zcol], axis=1)
    for t, (dy, dx) in enumerate(_TAPS):
        off = 1 + W + dy * W + dx
        sl = y_pad[:, off:off + N1]
        if dx == -1:
            sl = sl * mask_l
        elif dx == 1:
            sl = sl * mask_r
        contrib = (wdw_ref[:, t:t + 1] * sl).astype(jnp.bfloat16).astype(jnp.float32)
        if t == 0:
            z_ref[...] = contrib
        else:
            z_ref[...] += contrib

    q = z_ref[0:DIM]
    k = z_ref[DIM:2 * DIM]
    qsq = jnp.sum(q * q, axis=1, keepdims=True)
    ksq = jnp.sum(k * k, axis=1, keepdims=True)

    @pl.when(i == 0)
    def _init():
        qsq_ref[...] = qsq
        ksq_ref[...] = ksq

    @pl.when(i != 0)
    def _acc():
        qsq_ref[...] += qsq
        ksq_ref[...] += ksq


def _p2_body(qk_ref, qsq_ref, ksq_ref, gram_ref):
    i = pl.program_id(0)
    nq = jnp.maximum(jnp.sqrt(qsq_ref[...]), 1e-12)
    nk = jnp.maximum(jnp.sqrt(ksq_ref[...]), 1e-12)
    qh = (qk_ref[0:DIM] / nq).astype(jnp.bfloat16)
    kh = (qk_ref[DIM:2 * DIM] / nk).astype(jnp.bfloat16)
    g = jax.lax.dot_general(qh, kh, (((1,), (1,)), ((), ())),
                            preferred_element_type=jnp.float32)

    @pl.when(i == 0)
    def _init():
        gram_ref[...] = g

    @pl.when(i != 0)
    def _acc():
        gram_ref[...] += g


def _p3_body(gram_ref, temp_ref, a1_ref):
    KTOP = C // 2
    ii = jax.lax.broadcasted_iota(jnp.int32, (1, C, C), 1)
    jj = jax.lax.broadcasted_iota(jnp.int32, (1, C, C), 2)
    for h in range(HEADS):
        sl = slice(h * C, (h + 1) * C)
        ah = gram_ref[sl, sl] * temp_ref[h:h + 1, :]

        ai = ah[:, :, None]
        aj = ah[:, None, :]
        beats = jnp.logical_or(ai > aj, jnp.logical_and(ai == aj, ii < jj))
        rank = jnp.sum(beats.astype(jnp.float32), axis=1)
        keep = rank < KTOP

        masked = jnp.where(keep, ah, jnp.float32(-1e30))
        mx = jnp.max(masked, axis=-1, keepdims=True)
        e = jnp.where(keep, jnp.exp(ah - mx), 0.0)
        a1_ref[sl, :] = e / jnp.sum(e, axis=-1, keepdims=True)


def _p4_body(v_ref, a1_ref, wp_ref, a1p_ref, out_ref, o1_ref):
    a1b = a1_ref[...].astype(jnp.bfloat16)
    for h in range(HEADS):
        sl = slice(h * C, (h + 1) * C)
        o1_ref[sl, :] = jax.lax.dot_general(
            a1b[sl, :], v_ref[sl, :].astype(jnp.bfloat16),
            (((1,), (0,)), ((), ())), preferred_element_type=jnp.float32)
    t2 = (o1_ref[...] * a1p_ref[0, 0]).astype(jnp.bfloat16)
    out_ref[...] = jax.lax.dot_general(
        wp_ref[...].astype(jnp.bfloat16), t2,
        (((1,), (0,)), ((), ())), preferred_element_type=jnp.float32)


@jax.jit
def kernel(x, W_qkv, W_dw, W_proj, temperature, attn1p):
    xf = x.reshape(DIM, HWTOT)
    x3 = x.reshape(DIM, H, W)
    w1 = W_qkv.reshape(3 * DIM, DIM)
    wdw = W_dw.reshape(3 * DIM, 9)
    wp = W_proj.reshape(DIM, DIM)
    temp = temperature.reshape(HEADS, 1)
    a1p = attn1p.reshape(1, 1)

    z, qsq, ksq = pl.pallas_call(
        _p1_body,
        grid=(T1,),
        in_specs=[
            pl.BlockSpec((DIM, 8, W),
                         lambda i: (0, jnp.maximum((i * R1 - 1) // 8, 0), 0)),
            pl.BlockSpec((DIM, N1), lambda i: (0, i)),
            pl.BlockSpec((DIM, 8, W),
                         lambda i: (0, jnp.minimum(((i + 1) * R1) // 8, H // 8 - 1), 0)),
            pl.BlockSpec((3 * DIM, DIM), lambda i: (0, 0)),
            pl.BlockSpec((3 * DIM, 9), lambda i: (0, 0)),
        ],
        out_specs=[
            pl.BlockSpec((3 * DIM, N1), lambda i: (0, i)),
            pl.BlockSpec((DIM, 1), lambda i: (0, 0)),
            pl.BlockSpec((DIM, 1), lambda i: (0, 0)),
        ],
        out_shape=[
            jax.ShapeDtypeStruct((3 * DIM, HWTOT), jnp.float32),
            jax.ShapeDtypeStruct((DIM, 1), jnp.float32),
            jax.ShapeDtypeStruct((DIM, 1), jnp.float32),
        ],
    )(x3, xf, x3, w1, wdw)

    gram = pl.pallas_call(
        _p2_body,
        grid=(T2,),
        in_specs=[
            pl.BlockSpec((2 * DIM, N2), lambda i: (0, i)),
            pl.BlockSpec((DIM, 1), lambda i: (0, 0)),
            pl.BlockSpec((DIM, 1), lambda i: (0, 0)),
        ],
        out_specs=pl.BlockSpec((DIM, DIM), lambda i: (0, 0)),
        out_shape=jax.ShapeDtypeStruct((DIM, DIM), jnp.float32),
    )(z, qsq, ksq)

    a1 = pl.pallas_call(
        _p3_body,
        out_shape=jax.ShapeDtypeStruct((DIM, C), jnp.float32),
    )(gram, temp)

    out = pl.pallas_call(
        _p4_body,
        grid=(T4,),
        in_specs=[
            pl.BlockSpec((DIM, N4), lambda i: (2, i)),
            pl.BlockSpec((DIM, C), lambda i: (0, 0)),
            pl.BlockSpec((DIM, DIM), lambda i: (0, 0)),
            pl.BlockSpec((1, 1), lambda i: (0, 0)),
        ],
        out_specs=pl.BlockSpec((DIM, N4), lambda i: (0, i)),
        out_shape=jax.ShapeDtypeStruct((DIM, HWTOT), jnp.float32),
        scratch_shapes=[pltpu.VMEM((DIM, N4), jnp.float32)],
    )(z, a1, wp, a1p)

    return out.reshape(1, DIM, H, W)


# speedup vs baseline: 4.3725x; 1.0354x over previous
import jax
import jax.numpy as jnp
from jax.experimental import pallas as pl
from jax.experimental.pallas import tpu as pltpu

DIM = 192
HEADS = 6
C = DIM // HEADS
H = 224
W = 224
HWTOT = H * W

R1 = 16
T1 = H // R1
N1 = R1 * W
R2 = 16
T2 = H // R2
N2 = R2 * W
R4 = 16
T4 = H // R4
N4 = R4 * W

_TAPS = [(dy, dx) for dy in (-1, 0, 1) for dx in (-1, 0, 1)]


def _p1_body(x_top_ref, x_cur_ref, x_bot_ref, w1_ref, wdw_ref,
             zqk_ref, vb_ref, qsq_ref, ksq_ref, z_ref):
    i = pl.program_id(0)
    zero_row = jnp.zeros((DIM, W), jnp.float32)
    top = jnp.where(i == 0, zero_row, x_top_ref[:, 7, :])
    bot = jnp.where(i == T1 - 1, zero_row, x_bot_ref[:, 0, :])
    x_ext = jnp.concatenate([top, x_cur_ref[...], bot], axis=1)

    y_ext = jax.lax.dot_general(
        w1_ref[...].astype(jnp.bfloat16), x_ext.astype(jnp.bfloat16),
        (((1,), (0,)), ((), ())), preferred_element_type=jnp.float32)
    y_ext = y_ext.astype(jnp.bfloat16).astype(jnp.float32)

    widx = jax.lax.broadcasted_iota(jnp.int32, (1, N1), 1) % W
    mask_l = (widx != 0).astype(jnp.float32)
    mask_r = (widx != W - 1).astype(jnp.float32)
    zcol = jnp.zeros((3 * DIM, 1), jnp.float32)
    y_pad = jnp.concatenate([zcol, y_ext, zcol], axis=1)
    for t, (dy, dx) in enumerate(_TAPS):
        off = 1 + W + dy * W + dx
        sl = y_pad[:, off:off + N1]
        if dx == -1:
            sl = sl * mask_l
        elif dx == 1:
            sl = sl * mask_r
        contrib = (wdw_ref[:, t:t + 1] * sl).astype(jnp.bfloat16).astype(jnp.float32)
        if t == 0:
            z_ref[...] = contrib
        else:
            z_ref[...] += contrib

    q = z_ref[0:DIM]
    k = z_ref[DIM:2 * DIM]
    zqk_ref[...] = z_ref[0:2 * DIM]
    vb_ref[...] = z_ref[2 * DIM:3 * DIM].astype(jnp.bfloat16)
    qsq = jnp.sum(q * q, axis=1, keepdims=True)
    ksq = jnp.sum(k * k, axis=1, keepdims=True)

    @pl.when(i == 0)
    def _init():
        qsq_ref[...] = qsq
        ksq_ref[...] = ksq

    @pl.when(i != 0)
    def _acc():
        qsq_ref[...] += qsq
        ksq_ref[...] += ksq


def _p2_body(qk_ref, qsq_ref, ksq_ref, gram_ref):
    i = pl.program_id(0)
    nq = jnp.maximum(jnp.sqrt(qsq_ref[...]), 1e-12)
    nk = jnp.maximum(jnp.sqrt(ksq_ref[...]), 1e-12)
    qh = (qk_ref[0:DIM] / nq).astype(jnp.bfloat16)
    kh = (qk_ref[DIM:2 * DIM] / nk).astype(jnp.bfloat16)
    g = jax.lax.dot_general(qh, kh, (((1,), (1,)), ((), ())),
                            preferred_element_type=jnp.float32)

    @pl.when(i == 0)
    def _init():
        gram_ref[...] = g

    @pl.when(i != 0)
    def _acc():
        gram_ref[...] += g


def _p3_body(gram_ref, temp_ref, a1_ref):
    KTOP = C // 2
    ii = jax.lax.broadcasted_iota(jnp.int32, (1, C, C), 1)
    jj = jax.lax.broadcasted_iota(jnp.int32, (1, C, C), 2)
    for h in range(HEADS):
        sl = slice(h * C, (h + 1) * C)
        ah = gram_ref[sl, sl] * temp_ref[h:h + 1, :]

        ai = ah[:, :, None]
        aj = ah[:, None, :]
        beats = jnp.logical_or(ai > aj, jnp.logical_and(ai == aj, ii < jj))
        rank = jnp.sum(beats.astype(jnp.float32), axis=1)
        keep = rank < KTOP

        masked = jnp.where(keep, ah, jnp.float32(-1e30))
        mx = jnp.max(masked, axis=-1, keepdims=True)
        e = jnp.where(keep, jnp.exp(ah - mx), 0.0)
        a1_ref[sl, :] = e / jnp.sum(e, axis=-1, keepdims=True)


def _p4_body(v_ref, a1_ref, wp_ref, a1p_ref, out_ref, o1_ref):
    a1b = a1_ref[...].astype(jnp.bfloat16)
    for h in range(HEADS):
        sl = slice(h * C, (h + 1) * C)
        o1_ref[sl, :] = jax.lax.dot_general(
            a1b[sl, :], v_ref[sl, :],
            (((1,), (0,)), ((), ())), preferred_element_type=jnp.float32)
    t2 = (o1_ref[...] * a1p_ref[0, 0]).astype(jnp.bfloat16)
    out_ref[...] = jax.lax.dot_general(
        wp_ref[...].astype(jnp.bfloat16), t2,
        (((1,), (0,)), ((), ())), preferred_element_type=jnp.float32)


@jax.jit
def kernel(x, W_qkv, W_dw, W_proj, temperature, attn1p):
    xf = x.reshape(DIM, HWTOT)
    x3 = x.reshape(DIM, H, W)
    w1 = W_qkv.reshape(3 * DIM, DIM)
    wdw = W_dw.reshape(3 * DIM, 9)
    wp = W_proj.reshape(DIM, DIM)
    temp = temperature.reshape(HEADS, 1)
    a1p = attn1p.reshape(1, 1)

    zqk, vb, qsq, ksq = pl.pallas_call(
        _p1_body,
        grid=(T1,),
        in_specs=[
            pl.BlockSpec((DIM, 8, W),
                         lambda i: (0, jnp.maximum((i * R1 - 1) // 8, 0), 0)),
            pl.BlockSpec((DIM, N1), lambda i: (0, i)),
            pl.BlockSpec((DIM, 8, W),
                         lambda i: (0, jnp.minimum(((i + 1) * R1) // 8, H // 8 - 1), 0)),
            pl.BlockSpec((3 * DIM, DIM), lambda i: (0, 0)),
            pl.BlockSpec((3 * DIM, 9), lambda i: (0, 0)),
        ],
        out_specs=[
            pl.BlockSpec((2 * DIM, N1), lambda i: (0, i)),
            pl.BlockSpec((DIM, N1), lambda i: (0, i)),
            pl.BlockSpec((DIM, 1), lambda i: (0, 0)),
            pl.BlockSpec((DIM, 1), lambda i: (0, 0)),
        ],
        out_shape=[
            jax.ShapeDtypeStruct((2 * DIM, HWTOT), jnp.float32),
            jax.ShapeDtypeStruct((DIM, HWTOT), jnp.bfloat16),
            jax.ShapeDtypeStruct((DIM, 1), jnp.float32),
            jax.ShapeDtypeStruct((DIM, 1), jnp.float32),
        ],
        scratch_shapes=[pltpu.VMEM((3 * DIM, N1), jnp.float32)],
    )(x3, xf, x3, w1, wdw)

    gram = pl.pallas_call(
        _p2_body,
        grid=(T2,),
        in_specs=[
            pl.BlockSpec((2 * DIM, N2), lambda i: (0, i)),
            pl.BlockSpec((DIM, 1), lambda i: (0, 0)),
            pl.BlockSpec((DIM, 1), lambda i: (0, 0)),
        ],
        out_specs=pl.BlockSpec((DIM, DIM), lambda i: (0, 0)),
        out_shape=jax.ShapeDtypeStruct((DIM, DIM), jnp.float32),
    )(zqk, qsq, ksq)

    a1 = pl.pallas_call(
        _p3_body,
        out_shape=jax.ShapeDtypeStruct((DIM, C), jnp.float32),
    )(gram, temp)

    out = pl.pallas_call(
        _p4_body,
        grid=(T4,),
        in_specs=[
            pl.BlockSpec((DIM, N4), lambda i: (0, i)),
            pl.BlockSpec((DIM, C), lambda i: (0, 0)),
            pl.BlockSpec((DIM, DIM), lambda i: (0, 0)),
            pl.BlockSpec((1, 1), lambda i: (0, 0)),
        ],
        out_specs=pl.BlockSpec((DIM, N4), lambda i: (0, i)),
        out_shape=jax.ShapeDtypeStruct((DIM, HWTOT), jnp.float32),
        scratch_shapes=[pltpu.VMEM((DIM, N4), jnp.float32)],
    )(vb, a1, wp, a1p)

    return out.reshape(1, DIM, H, W)
